# baseline (speedup 1.0000x reference)
import functools

import jax
import jax.numpy as jnp
from jax import lax
from jax.experimental import pallas as pl
from jax.experimental.pallas import tpu as pltpu
from jax.experimental.pallas import tpu_sc as plsc

EMB = 128
VOCAB1 = 100001
UTT = 20
BATCH = 4096
NOUT = 26 * 64

NC = 2
NS = 16
NW = NC * NS
PERW = BATCH // NW
NB = 16
NCH = PERW // NB
LANES = 16


def _sc_body(utts_hbm, table_hbm, embs_hbm, idx_v, rows_v, out_v, sem):
    wid = lax.axis_index("s") * NC + lax.axis_index("c")
    base = wid * PERW

    pltpu.sync_copy(utts_hbm.at[:, pl.ds(base, PERW)], idx_v)
    for p in range(UTT):
        for j in range(PERW // LANES):
            sl = pl.ds(j * LANES, LANES)
            idx_v[p, sl] = idx_v[p, sl] + (p * VOCAB1)

    def chunk_body(ci, carry):
        cps = []
        for p in range(UTT):
            cp = pltpu.async_copy(
                table_hbm.at[idx_v.at[p, pl.ds(ci * NB, NB)]],
                rows_v.at[p], sem)
            cps.append(cp)
        for cp in cps:
            cp.wait()

        def bbody(b, c2):
            for c in range(EMB // LANES):
                sl = pl.ds(c * LANES, LANES)
                acc = rows_v[0, b, sl]
                for p in range(1, UTT):
                    acc = acc + rows_v[p, b, sl]
                out_v[ci * NB + b, sl] = acc
            return c2

        lax.fori_loop(0, NB, bbody, 0)
        return carry

    lax.fori_loop(0, NCH, chunk_body, 0)
    pltpu.sync_copy(out_v, embs_hbm.at[pl.ds(base, PERW)])


@functools.partial(
    pl.kernel,
    mesh=plsc.VectorSubcoreMesh(core_axis_name="c", subcore_axis_name="s"),
    out_type=jax.ShapeDtypeStruct((BATCH, EMB), jnp.float32),
    scratch_types=[
        pltpu.VMEM((UTT, PERW), jnp.int32),
        pltpu.VMEM((UTT, NB, EMB), jnp.float32),
        pltpu.VMEM((PERW, EMB), jnp.float32),
        pltpu.SemaphoreType.DMA,
    ],
)
def _sc_gather_sum(utts_hbm, table_hbm, embs_hbm, idx_v, rows_v, out_v, sem):
    _sc_body(utts_hbm, table_hbm, embs_hbm, idx_v, rows_v, out_v, sem)


def _tc_body(e_ref, w_ref, b_ref, o_ref):
    x = jnp.tanh(e_ref[...])
    o_ref[...] = (
        jnp.dot(x, w_ref[...], preferred_element_type=jnp.float32) + b_ref[...]
    )


_TB = 512


def _tc_dense(embs, W2, b2):
    return pl.pallas_call(
        _tc_body,
        grid=(BATCH // _TB,),
        in_specs=[
            pl.BlockSpec((_TB, EMB), lambda i: (i, 0)),
            pl.BlockSpec((EMB, NOUT), lambda i: (0, 0)),
            pl.BlockSpec((1, NOUT), lambda i: (0, 0)),
        ],
        out_specs=pl.BlockSpec((_TB, NOUT), lambda i: (i, 0)),
        out_shape=jax.ShapeDtypeStruct((BATCH, NOUT), jnp.float32),
    )(embs, W2, b2)


def kernel(utts, emb_table, W2, b2):
    embs = _sc_gather_sum(utts, emb_table)
    x = _tc_dense(embs, W2, b2.reshape(1, NOUT))
    return x.reshape(BATCH, 26, 64)
